# baseline (speedup 1.0000x reference)
import jax
import jax.numpy as jnp
from jax import lax
from jax.experimental import pallas as pl
from jax.experimental.pallas import tpu as pltpu
from jax.experimental.pallas import tpu_sc as plsc

_W = 1000.0
NC, NS, L = 2, 16, 16
NW = NC * NS
FACES = 256 * 512
FPW = FACES // NW
BLK = 2048
ROWS_PB = BLK // 512
CHUNKS = 512 // L


def _sc_body(ff8_hbm, fv8_hbm, padded_hbm, out_hbm,
             idxb, vids8, v0, v1, v2, pb0, pb1, pb2, obuf, sem):
  vids = (v0, v1, v2)
  pbuf = (pb0, pb1, pb2)
  wid = lax.axis_index("s") * NC + lax.axis_index("c")
  base = wid * FPW

  loss = jnp.float32(0.0)
  for blk in range(FPW // BLK):
    pltpu.sync_copy(padded_hbm.at[pl.ds(base + blk * BLK, BLK)], idxb)

    pltpu.async_copy(ff8_hbm.at[idxb], vids8, sem).wait()

    def t_body(j, carry):
      rows = j * L + lax.iota(jnp.int32, L)
      for k in range(3):
        vids[k][pl.ds(j * L, L)] = plsc.load_gather(
            vids8, [rows, jnp.full((L,), k, jnp.int32)])
      return carry

    lax.fori_loop(0, BLK // L, t_body, 0, unroll=2)

    copies = [pltpu.make_async_copy(fv8_hbm.at[vids[k]], pbuf[k], sem)
              for k in range(3)]
    for c in copies:
      c.start()
    for c in copies:
      c.wait()

    def row_body(r, loss_acc):
      def chunk_body(j, acc):
        rows = (r * CHUNKS + j) * L + lax.iota(jnp.int32, L)
        p = []
        for k in range(3):
          p.append([
              plsc.load_gather(pbuf[k],
                               [rows, jnp.full((L,), c, jnp.int32)])
              for c in range(3)
          ])
        (x0, y0, z0), (x1, y1, z1), (x2, y2, z2) = p
        sv = (x0 * (y1 * z2 - z1 * y2)
              + y0 * (z1 * x2 - x1 * z2)
              + z0 * (x1 * y2 - y1 * x2))
        return acc + sv

      acc = lax.fori_loop(0, CHUNKS, chunk_body,
                          jnp.zeros((L,), jnp.float32), unroll=2)
      return loss_acc + jnp.abs(jnp.sum(acc))

    loss = lax.fori_loop(0, ROWS_PB, row_body, loss)

  lane = lax.iota(jnp.int32, L)
  obuf[...] = jnp.where(lane == 0, loss * _W, jnp.float32(0.0))
  pltpu.sync_copy(obuf, out_hbm.at[wid])


@jax.jit
def kernel(final_v, ff, padded_tensor):
  ff8 = jnp.pad(ff, ((0, 0), (0, 5)))
  fv8 = jnp.pad(final_v, ((0, 0), (0, 5)))
  padded_flat = padded_tensor.reshape(-1)
  mesh = plsc.VectorSubcoreMesh(core_axis_name="c", subcore_axis_name="s")
  partials = pl.kernel(
      _sc_body,
      out_type=jax.ShapeDtypeStruct((NW, L), jnp.float32),
      mesh=mesh,
      scratch_types=[
          pltpu.VMEM((BLK,), jnp.int32),
          pltpu.VMEM((BLK, 8), jnp.int32),
          pltpu.VMEM((BLK,), jnp.int32),
          pltpu.VMEM((BLK,), jnp.int32),
          pltpu.VMEM((BLK,), jnp.int32),
          pltpu.VMEM((BLK, 8), jnp.float32),
          pltpu.VMEM((BLK, 8), jnp.float32),
          pltpu.VMEM((BLK, 8), jnp.float32),
          pltpu.VMEM((L,), jnp.float32),
          pltpu.SemaphoreType.DMA,
      ],
      compiler_params=pltpu.CompilerParams(
          needs_layout_passes=False, use_tc_tiling_on_sc=False),
  )(ff8, fv8, padded_flat)
  return jnp.sum(partials)


# speedup vs baseline: 3.6989x; 3.6989x over previous
import jax
import jax.numpy as jnp
from jax import lax
from jax.experimental import pallas as pl
from jax.experimental.pallas import tpu as pltpu
from jax.experimental.pallas import tpu_sc as plsc

_W = 1000.0
NC, NS, L = 2, 16, 16
NW = NC * NS
NV = 65536
VPT = NV // NS
FACES = 256 * 512
FPW = FACES // NW
BLK = 2048
ROWS_PB = BLK // 512
CHUNKS = 512 // L


def _sc_body(vx, vy, vz, f0, f1, f2, padded_hbm, out_hbm,
             fv8_hbm, cbuf, bb, idxb, v0, v1, v2, pb0, pb1, pb2, obuf, sem):
  ftabs = (f0, f1, f2)
  vtabs = (vx, vy, vz)
  vids = (v0, v1, v2)
  pbuf = (pb0, pb1, pb2)
  cid = lax.axis_index("c")
  sid = lax.axis_index("s")
  wid = sid * NC + cid
  base = wid * FPW

  for c in range(3):
    pltpu.sync_copy(vtabs[c].at[pl.ds(sid * VPT, VPT)],
                    cbuf.at[pl.ds(c * VPT, VPT)])

  def pack_body(j, carry):
    rows = j * L + lax.iota(jnp.int32, L)
    for c in range(3):
      x = cbuf[pl.ds(c * VPT + j * L, L)]
      plsc.store_scatter(bb, [rows, jnp.full((L,), c, jnp.int32)], x)
    return carry

  lax.fori_loop(0, VPT // L, pack_body, 0, unroll=2)
  pltpu.sync_copy(bb, fv8_hbm.at[pl.ds(sid * VPT, VPT)])
  plsc.subcore_barrier()

  loss = jnp.float32(0.0)
  for blk in range(FPW // BLK):
    pltpu.sync_copy(padded_hbm.at[pl.ds(base + blk * BLK, BLK)], idxb)

    copies = [pltpu.make_async_copy(ftabs[k].at[idxb], vids[k], sem)
              for k in range(3)]
    for c in copies:
      c.start()
    for c in copies:
      c.wait()

    copies = [pltpu.make_async_copy(fv8_hbm.at[vids[k]], pbuf[k], sem)
              for k in range(3)]
    for c in copies:
      c.start()
    for c in copies:
      c.wait()

    def row_body(r, loss_acc):
      def chunk_body(j, acc):
        rows = (r * CHUNKS + j) * L + lax.iota(jnp.int32, L)
        p = []
        for k in range(3):
          p.append([
              plsc.load_gather(pbuf[k],
                               [rows, jnp.full((L,), c, jnp.int32)])
              for c in range(3)
          ])
        (x0, y0, z0), (x1, y1, z1), (x2, y2, z2) = p
        sv = (x0 * (y1 * z2 - z1 * y2)
              + y0 * (z1 * x2 - x1 * z2)
              + z0 * (x1 * y2 - y1 * x2))
        return acc + sv

      acc = lax.fori_loop(0, CHUNKS, chunk_body,
                          jnp.zeros((L,), jnp.float32), unroll=2)
      return loss_acc + jnp.abs(jnp.sum(acc))

    loss = lax.fori_loop(0, ROWS_PB, row_body, loss)

  lane = lax.iota(jnp.int32, L)
  obuf[...] = jnp.where(lane == 0, loss * _W, jnp.float32(0.0))
  pltpu.sync_copy(obuf, out_hbm.at[wid])


@jax.jit
def kernel(final_v, ff, padded_tensor):
  fvT = final_v.T
  ffT = ff.T
  vx, vy, vz = fvT[0], fvT[1], fvT[2]
  f0, f1, f2 = ffT[0], ffT[1], ffT[2]
  padded_flat = padded_tensor.reshape(-1)
  mesh = plsc.VectorSubcoreMesh(core_axis_name="c", subcore_axis_name="s")
  partials = pl.kernel(
      _sc_body,
      out_type=jax.ShapeDtypeStruct((NW, L), jnp.float32),
      mesh=mesh,
      scratch_types=[
          pltpu.HBM((NV, 8), jnp.float32),
          pltpu.VMEM((3 * VPT,), jnp.float32),
          pltpu.VMEM((VPT, 8), jnp.float32),
          pltpu.VMEM((BLK,), jnp.int32),
          pltpu.VMEM((BLK,), jnp.int32),
          pltpu.VMEM((BLK,), jnp.int32),
          pltpu.VMEM((BLK,), jnp.int32),
          pltpu.VMEM((BLK, 8), jnp.float32),
          pltpu.VMEM((BLK, 8), jnp.float32),
          pltpu.VMEM((BLK, 8), jnp.float32),
          pltpu.VMEM((L,), jnp.float32),
          pltpu.SemaphoreType.DMA,
      ],
      compiler_params=pltpu.CompilerParams(
          needs_layout_passes=False, use_tc_tiling_on_sc=False),
  )(vx, vy, vz, f0, f1, f2, padded_flat)
  return jnp.sum(partials)


# speedup vs baseline: 4.0520x; 1.0955x over previous
import jax
import jax.numpy as jnp
from jax import lax
from jax.experimental import pallas as pl
from jax.experimental.pallas import tpu as pltpu
from jax.experimental.pallas import tpu_sc as plsc

_W = 1000.0
NC, NS, L = 2, 16, 16
NW = NC * NS
NV = 65536
VPT = NV // NS
FACES = 256 * 512
FPW = FACES // NW
BLK = 1024
NBLK = FPW // BLK
ROWS_PB = 512 // BLK if BLK >= 512 else 0
CHUNKS = BLK // L
ROW_CHUNKS = 512 // L


def _sc_body(vx, vy, vz, f0, f1, f2, padded_hbm, out_hbm,
             fv8_hbm, cbuf, bb, idxb,
             v00, v01, v02, v10, v11, v12, v20, v21, v22, v30, v31, v32,
             pa0, pa1, pa2, pb0, pb1, pb2, obuf, sem, semv):
  ftabs = (f0, f1, f2)
  vtabs = (vx, vy, vz)
  vids = ((v00, v01, v02), (v10, v11, v12),
          (v20, v21, v22), (v30, v31, v32))
  pbufs = ((pa0, pa1, pa2), (pb0, pb1, pb2))
  cid = lax.axis_index("c")
  sid = lax.axis_index("s")
  wid = sid * NC + cid
  base = wid * FPW

  pltpu.sync_copy(padded_hbm.at[pl.ds(base, FPW)], idxb)
  id_copies = []
  for b in range(NBLK):
    for k in range(3):
      c = pltpu.make_async_copy(
          ftabs[k].at[idxb.at[pl.ds(b * BLK, BLK)]], vids[b][k], semv)
      c.start()
      id_copies.append(c)

  for c in range(3):
    pltpu.sync_copy(vtabs[c].at[pl.ds(sid * VPT, VPT)],
                    cbuf.at[pl.ds(c * VPT, VPT)])

  def pack_body(j, carry):
    rows = j * L + lax.iota(jnp.int32, L)
    for c in range(3):
      x = cbuf[pl.ds(c * VPT + j * L, L)]
      plsc.store_scatter(bb, [rows, jnp.full((L,), c, jnp.int32)], x)
    return carry

  lax.fori_loop(0, VPT // L, pack_body, 0, unroll=2)
  pltpu.sync_copy(bb, fv8_hbm.at[pl.ds(sid * VPT, VPT)])
  plsc.subcore_barrier()

  for c in id_copies:
    c.wait()

  def fire(b):
    cs = [pltpu.make_async_copy(fv8_hbm.at[vids[b][k]], pbufs[b % 2][k], sem)
          for k in range(3)]
    for c in cs:
      c.start()
    return cs

  def drain(cs):
    for c in cs:
      c.wait()

  def compute(b, loss):
    pbuf = pbufs[b % 2]

    def row_body(r, loss_acc):
      def chunk_body(j, acc):
        rows = (r * ROW_CHUNKS + j) * L + lax.iota(jnp.int32, L)
        p = []
        for k in range(3):
          p.append([
              plsc.load_gather(pbuf[k],
                               [rows, jnp.full((L,), c, jnp.int32)])
              for c in range(3)
          ])
        (x0, y0, z0), (x1, y1, z1), (x2, y2, z2) = p
        sv = (x0 * (y1 * z2 - z1 * y2)
              + y0 * (z1 * x2 - x1 * z2)
              + z0 * (x1 * y2 - y1 * x2))
        return acc + sv

      acc = lax.fori_loop(0, ROW_CHUNKS, chunk_body,
                          jnp.zeros((L,), jnp.float32), unroll=2)
      return loss_acc + jnp.abs(jnp.sum(acc))

    return lax.fori_loop(0, BLK // 512, row_body, loss)

  loss = jnp.float32(0.0)
  inflight = fire(0)
  for b in range(NBLK):
    drain(inflight)
    if b + 1 < NBLK:
      nxt = fire(b + 1)
    loss = compute(b, loss)
    if b + 1 < NBLK:
      inflight = nxt

  lane = lax.iota(jnp.int32, L)
  obuf[...] = jnp.where(lane == 0, loss * _W, jnp.float32(0.0))
  pltpu.sync_copy(obuf, out_hbm.at[wid])


@jax.jit
def kernel(final_v, ff, padded_tensor):
  fvT = final_v.T
  ffT = ff.T
  vx, vy, vz = fvT[0], fvT[1], fvT[2]
  f0, f1, f2 = ffT[0], ffT[1], ffT[2]
  padded_flat = padded_tensor.reshape(-1)
  mesh = plsc.VectorSubcoreMesh(core_axis_name="c", subcore_axis_name="s")
  iblk = pltpu.VMEM((BLK,), jnp.int32)
  fblk8 = pltpu.VMEM((BLK, 8), jnp.float32)
  partials = pl.kernel(
      _sc_body,
      out_type=jax.ShapeDtypeStruct((NW, L), jnp.float32),
      mesh=mesh,
      scratch_types=(
          [pltpu.HBM((NV, 8), jnp.float32),
           pltpu.VMEM((3 * VPT,), jnp.float32),
           pltpu.VMEM((VPT, 8), jnp.float32),
           pltpu.VMEM((FPW,), jnp.int32)]
          + [iblk] * 12 + [fblk8] * 6
          + [pltpu.VMEM((L,), jnp.float32),
             pltpu.SemaphoreType.DMA, pltpu.SemaphoreType.DMA]
      ),
      compiler_params=pltpu.CompilerParams(
          needs_layout_passes=False, use_tc_tiling_on_sc=False),
  )(vx, vy, vz, f0, f1, f2, padded_flat)
  return jnp.sum(partials)


# speedup vs baseline: 4.2827x; 1.0569x over previous
import jax
import jax.numpy as jnp
from jax import lax
from jax.experimental import pallas as pl
from jax.experimental.pallas import tpu as pltpu
from jax.experimental.pallas import tpu_sc as plsc

_W = 1000.0
NC, NS, L = 2, 16, 16
NW = NC * NS
NV = 65536
VPT = NV // NS
FACES = 256 * 512
FPW = FACES // NW
BLK = 1024
NBLK = FPW // BLK
ROW_CHUNKS = 512 // L

_PARAMS = dict(
    compiler_params=pltpu.CompilerParams(
        needs_layout_passes=False, use_tc_tiling_on_sc=False),
)


def _pack_body(vx, vy, vz, fv8_hbm, cbuf, bb, sem):
  del sem
  vtabs = (vx, vy, vz)
  sid = lax.axis_index("s")

  for c in range(3):
    pltpu.sync_copy(vtabs[c].at[pl.ds(sid * VPT, VPT)],
                    cbuf.at[pl.ds(c * VPT, VPT)])

  def pack_body(j, carry):
    rows = j * L + lax.iota(jnp.int32, L)
    for c in range(3):
      x = cbuf[pl.ds(c * VPT + j * L, L)]
      plsc.store_scatter(bb, [rows, jnp.full((L,), c, jnp.int32)], x)
    return carry

  lax.fori_loop(0, VPT // L, pack_body, 0, unroll=2)
  pltpu.sync_copy(bb, fv8_hbm.at[pl.ds(sid * VPT, VPT)])


def _main_body(fv8_hbm, f0, f1, f2, padded_hbm, out_hbm,
               idxb,
               v00, v01, v02, v10, v11, v12, v20, v21, v22, v30, v31, v32,
               pa0, pa1, pa2, pb0, pb1, pb2, obuf, sem, semv):
  ftabs = (f0, f1, f2)
  vids = ((v00, v01, v02), (v10, v11, v12),
          (v20, v21, v22), (v30, v31, v32))
  pbufs = ((pa0, pa1, pa2), (pb0, pb1, pb2))
  cid = lax.axis_index("c")
  sid = lax.axis_index("s")
  wid = sid * NC + cid
  base = wid * FPW

  pltpu.sync_copy(padded_hbm.at[pl.ds(base, FPW)], idxb)
  id_copies = []
  for b in range(NBLK):
    for k in range(3):
      c = pltpu.make_async_copy(
          ftabs[k].at[idxb.at[pl.ds(b * BLK, BLK)]], vids[b][k], semv)
      c.start()
      id_copies.append(c)
  for c in id_copies:
    c.wait()

  def fire(b):
    cs = [pltpu.make_async_copy(fv8_hbm.at[vids[b][k]], pbufs[b % 2][k], sem)
          for k in range(3)]
    for c in cs:
      c.start()
    return cs

  def drain(cs):
    for c in cs:
      c.wait()

  def compute(b, loss):
    pbuf = pbufs[b % 2]

    def row_body(r, loss_acc):
      def chunk_body(j, acc):
        rows = (r * ROW_CHUNKS + j) * L + lax.iota(jnp.int32, L)
        p = []
        for k in range(3):
          p.append([
              plsc.load_gather(pbuf[k],
                               [rows, jnp.full((L,), c, jnp.int32)])
              for c in range(3)
          ])
        (x0, y0, z0), (x1, y1, z1), (x2, y2, z2) = p
        sv = (x0 * (y1 * z2 - z1 * y2)
              + y0 * (z1 * x2 - x1 * z2)
              + z0 * (x1 * y2 - y1 * x2))
        return acc + sv

      acc = lax.fori_loop(0, ROW_CHUNKS, chunk_body,
                          jnp.zeros((L,), jnp.float32), unroll=2)
      return loss_acc + jnp.abs(jnp.sum(acc))

    return lax.fori_loop(0, BLK // 512, row_body, loss)

  loss = jnp.float32(0.0)
  inflight = fire(0)
  for b in range(NBLK):
    drain(inflight)
    if b + 1 < NBLK:
      nxt = fire(b + 1)
    loss = compute(b, loss)
    if b + 1 < NBLK:
      inflight = nxt

  lane = lax.iota(jnp.int32, L)
  obuf[...] = jnp.where(lane == 0, loss * _W, jnp.float32(0.0))
  pltpu.sync_copy(obuf, out_hbm.at[wid])


@jax.jit
def kernel(final_v, ff, padded_tensor):
  fvT = final_v.T
  ffT = ff.T
  vx, vy, vz = fvT[0], fvT[1], fvT[2]
  f0, f1, f2 = ffT[0], ffT[1], ffT[2]
  padded_flat = padded_tensor.reshape(-1)
  mesh = plsc.VectorSubcoreMesh(core_axis_name="c", subcore_axis_name="s")

  fv8 = pl.kernel(
      _pack_body,
      out_type=jax.ShapeDtypeStruct((NV, 8), jnp.float32),
      mesh=mesh,
      scratch_types=[
          pltpu.VMEM((3 * VPT,), jnp.float32),
          pltpu.VMEM((VPT, 8), jnp.float32),
          pltpu.SemaphoreType.DMA,
      ],
      **_PARAMS,
  )(vx, vy, vz)

  iblk = pltpu.VMEM((BLK,), jnp.int32)
  fblk8 = pltpu.VMEM((BLK, 8), jnp.float32)
  partials = pl.kernel(
      _main_body,
      out_type=jax.ShapeDtypeStruct((NW, L), jnp.float32),
      mesh=mesh,
      scratch_types=(
          [pltpu.VMEM((FPW,), jnp.int32)]
          + [iblk] * 12 + [fblk8] * 6
          + [pltpu.VMEM((L,), jnp.float32),
             pltpu.SemaphoreType.DMA, pltpu.SemaphoreType.DMA]
      ),
      **_PARAMS,
  )(fv8, f0, f1, f2, padded_flat)
  return jnp.sum(partials)


# speedup vs baseline: 4.3487x; 1.0154x over previous
import jax
import jax.numpy as jnp
from jax import lax
from jax.experimental import pallas as pl
from jax.experimental.pallas import tpu as pltpu
from jax.experimental.pallas import tpu_sc as plsc

_W = 1000.0
NC, NS, L = 2, 16, 16
NW = NC * NS
NV = 65536
VPT = NV // NS
FACES = 256 * 512
FPW = FACES // NW
BLK = 2048
NBLK = FPW // BLK
ROW_CHUNKS = 512 // L

_PARAMS = dict(
    compiler_params=pltpu.CompilerParams(
        needs_layout_passes=False, use_tc_tiling_on_sc=False),
)


def _pack_body(vx, vy, vz, fv8_hbm, cbuf, bb, sem):
  del sem
  vtabs = (vx, vy, vz)
  sid = lax.axis_index("s")

  for c in range(3):
    pltpu.sync_copy(vtabs[c].at[pl.ds(sid * VPT, VPT)],
                    cbuf.at[pl.ds(c * VPT, VPT)])

  def pack_body(j, carry):
    rows = j * L + lax.iota(jnp.int32, L)
    for c in range(3):
      x = cbuf[pl.ds(c * VPT + j * L, L)]
      plsc.store_scatter(bb, [rows, jnp.full((L,), c, jnp.int32)], x)
    return carry

  lax.fori_loop(0, VPT // L, pack_body, 0, unroll=2)
  pltpu.sync_copy(bb, fv8_hbm.at[pl.ds(sid * VPT, VPT)])


def _main_body(fv8_hbm, f0, f1, f2, padded_hbm, out_hbm,
               idxb,
               v00, v01, v02, v10, v11, v12,
               pa0, pa1, pa2, pb0, pb1, pb2, obuf, sem, semv):
  ftabs = (f0, f1, f2)
  vids = ((v00, v01, v02), (v10, v11, v12))
  pbufs = ((pa0, pa1, pa2), (pb0, pb1, pb2))
  cid = lax.axis_index("c")
  sid = lax.axis_index("s")
  wid = sid * NC + cid
  base = wid * FPW

  pltpu.sync_copy(padded_hbm.at[pl.ds(base, FPW)], idxb)
  id_copies = []
  for b in range(NBLK):
    for k in range(3):
      c = pltpu.make_async_copy(
          ftabs[k].at[idxb.at[pl.ds(b * BLK, BLK)]], vids[b][k], semv)
      c.start()
      id_copies.append(c)
  for c in id_copies:
    c.wait()

  def fire(b):
    cs = [pltpu.make_async_copy(fv8_hbm.at[vids[b][k]], pbufs[b % 2][k], sem)
          for k in range(3)]
    for c in cs:
      c.start()
    return cs

  def drain(cs):
    for c in cs:
      c.wait()

  def compute(b, loss):
    pbuf = pbufs[b % 2]

    def row_body(r, loss_acc):
      def chunk_body(j, acc):
        rows = (r * ROW_CHUNKS + j) * L + lax.iota(jnp.int32, L)
        p = []
        for k in range(3):
          p.append([
              plsc.load_gather(pbuf[k],
                               [rows, jnp.full((L,), c, jnp.int32)])
              for c in range(3)
          ])
        (x0, y0, z0), (x1, y1, z1), (x2, y2, z2) = p
        sv = (x0 * (y1 * z2 - z1 * y2)
              + y0 * (z1 * x2 - x1 * z2)
              + z0 * (x1 * y2 - y1 * x2))
        return acc + sv

      acc = lax.fori_loop(0, ROW_CHUNKS, chunk_body,
                          jnp.zeros((L,), jnp.float32), unroll=2)
      return loss_acc + jnp.abs(jnp.sum(acc))

    return lax.fori_loop(0, BLK // 512, row_body, loss)

  loss = jnp.float32(0.0)
  inflight = fire(0)
  for b in range(NBLK):
    drain(inflight)
    if b + 1 < NBLK:
      nxt = fire(b + 1)
    loss = compute(b, loss)
    if b + 1 < NBLK:
      inflight = nxt

  lane = lax.iota(jnp.int32, L)
  obuf[...] = jnp.where(lane == 0, loss * _W, jnp.float32(0.0))
  pltpu.sync_copy(obuf, out_hbm.at[wid])


@jax.jit
def kernel(final_v, ff, padded_tensor):
  fvT = final_v.T
  ffT = ff.T
  vx, vy, vz = fvT[0], fvT[1], fvT[2]
  f0, f1, f2 = ffT[0], ffT[1], ffT[2]
  padded_flat = padded_tensor.reshape(-1)
  mesh = plsc.VectorSubcoreMesh(core_axis_name="c", subcore_axis_name="s")

  fv8 = pl.kernel(
      _pack_body,
      out_type=jax.ShapeDtypeStruct((NV, 8), jnp.float32),
      mesh=mesh,
      scratch_types=[
          pltpu.VMEM((3 * VPT,), jnp.float32),
          pltpu.VMEM((VPT, 8), jnp.float32),
          pltpu.SemaphoreType.DMA,
      ],
      **_PARAMS,
  )(vx, vy, vz)

  iblk = pltpu.VMEM((BLK,), jnp.int32)
  fblk8 = pltpu.VMEM((BLK, 8), jnp.float32)
  partials = pl.kernel(
      _main_body,
      out_type=jax.ShapeDtypeStruct((NW, L), jnp.float32),
      mesh=mesh,
      scratch_types=(
          [pltpu.VMEM((FPW,), jnp.int32)]
          + [iblk] * 6 + [fblk8] * 6
          + [pltpu.VMEM((L,), jnp.float32),
             pltpu.SemaphoreType.DMA, pltpu.SemaphoreType.DMA]
      ),
      **_PARAMS,
  )(fv8, f0, f1, f2, padded_flat)
  return jnp.sum(partials)
